# baseline (speedup 1.0000x reference)
import functools
import math

import jax
import jax.numpy as jnp
from jax import lax
from jax.experimental import pallas as pl
from jax.experimental.pallas import tpu as pltpu
from jax.experimental.pallas import tpu_sc as plsc

D_MODEL = 64
_SCALE = math.sqrt(D_MODEL)


@functools.lru_cache(maxsize=None)
def _make_sc_gather(batch: int):
    info = plsc.get_sparse_core_info()
    num_cores, num_subcores = info.num_cores, info.num_subcores
    num_workers = num_cores * num_subcores
    assert batch % (8 * num_workers) == 0
    b_per_w = batch // num_workers
    chunk = 512
    while b_per_w % chunk != 0:
        chunk //= 2
    n_chunks = b_per_w // chunk
    mesh = plsc.VectorSubcoreMesh(core_axis_name="c", subcore_axis_name="s")

    @functools.partial(
        pl.kernel,
        mesh=mesh,
        out_type=jax.ShapeDtypeStruct((batch, D_MODEL), jnp.float32),
        scratch_types=[
            pltpu.VMEM((chunk,), jnp.int32),
            pltpu.VMEM((chunk, D_MODEL), jnp.float32),
            pltpu.SemaphoreType.DMA,
        ],
        compiler_params=pltpu.CompilerParams(use_tc_tiling_on_sc=False),
    )
    def sc_kernel(idx_hbm, table_hbm, out_hbm, idx_v, rows_v, sem):
        wid = lax.axis_index("s") * num_cores + lax.axis_index("c")
        base = wid * b_per_w

        def do_chunk(g, carry):
            off = base + g * chunk
            pltpu.sync_copy(idx_hbm.at[pl.ds(off, chunk)], idx_v)
            pltpu.async_copy(table_hbm.at[idx_v], rows_v, sem).wait()

            def scale_row(i, c):
                for j in range(D_MODEL // 16):
                    sl = pl.ds(j * 16, 16)
                    rows_v[i, sl] = rows_v[i, sl] * _SCALE
                return c

            lax.fori_loop(0, chunk, scale_row, 0)
            pltpu.sync_copy(rows_v, out_hbm.at[pl.ds(off, chunk)])
            return carry

        lax.fori_loop(0, n_chunks, do_chunk, 0)

    return sc_kernel


def kernel(x, table):
    b, s = x.shape
    flat_idx = x.reshape(b * s).astype(jnp.int32)
    out = _make_sc_gather(b * s)(flat_idx, table)
    return out.reshape(b, s, D_MODEL)


# speedup vs baseline: 1.1320x; 1.1320x over previous
import functools
import math

import jax
import jax.numpy as jnp
from jax import lax
from jax.experimental import pallas as pl
from jax.experimental.pallas import tpu as pltpu
from jax.experimental.pallas import tpu_sc as plsc

D_MODEL = 64
_SCALE = math.sqrt(D_MODEL)
_CHUNK = 512


@functools.lru_cache(maxsize=None)
def _make_sc_gather(batch: int):
    info = plsc.get_sparse_core_info()
    num_cores, num_subcores = info.num_cores, info.num_subcores
    num_workers = num_cores * num_subcores
    assert batch % (num_workers * _CHUNK) == 0
    b_per_w = batch // num_workers
    n_chunks = b_per_w // _CHUNK
    assert n_chunks % 2 == 0
    mesh = plsc.VectorSubcoreMesh(core_axis_name="c", subcore_axis_name="s")

    @functools.partial(
        pl.kernel,
        mesh=mesh,
        out_type=jax.ShapeDtypeStruct((batch, D_MODEL), jnp.float32),
        scratch_types=[
            pltpu.VMEM((b_per_w,), jnp.int32),
            pltpu.VMEM((_CHUNK, D_MODEL), jnp.float32),
            pltpu.VMEM((_CHUNK, D_MODEL), jnp.float32),
            pltpu.SemaphoreType.DMA,
            pltpu.SemaphoreType.DMA,
            pltpu.SemaphoreType.DMA,
            pltpu.SemaphoreType.DMA,
        ],
        compiler_params=pltpu.CompilerParams(use_tc_tiling_on_sc=False),
    )
    def sc_kernel(idx_hbm, table_hbm, out_hbm, idx_v, rows0, rows1,
                  gsem0, gsem1, ssem0, ssem1):
        wid = lax.axis_index("s") * num_cores + lax.axis_index("c")
        base = wid * b_per_w
        rows = (rows0, rows1)
        gsem = (gsem0, gsem1)
        ssem = (ssem0, ssem1)

        pltpu.sync_copy(idx_hbm.at[pl.ds(base, b_per_w)], idx_v)

        pltpu.async_copy(table_hbm.at[idx_v.at[pl.ds(0, _CHUNK)]], rows0, gsem0)

        def half_step(g, b):
            @pl.when(g + 1 < n_chunks)
            def _():
                @pl.when(g >= 1)
                def _():
                    pltpu.make_async_copy(
                        rows[b ^ 1],
                        out_hbm.at[pl.ds(base, _CHUNK)],
                        ssem[b ^ 1],
                    ).wait()

                pltpu.async_copy(
                    table_hbm.at[idx_v.at[pl.ds((g + 1) * _CHUNK, _CHUNK)]],
                    rows[b ^ 1],
                    gsem[b ^ 1],
                )

            pltpu.make_async_copy(
                table_hbm.at[idx_v.at[pl.ds(g * _CHUNK, _CHUNK)]], rows[b], gsem[b]
            ).wait()

            @plsc.parallel_loop(0, _CHUNK, unroll=8)
            def _(i):
                for j in range(D_MODEL // 16):
                    sl = pl.ds(j * 16, 16)
                    rows[b][i, sl] = rows[b][i, sl] * _SCALE

            pltpu.async_copy(
                rows[b], out_hbm.at[pl.ds(base + g * _CHUNK, _CHUNK)], ssem[b]
            )

        def do_pair(p, carry):
            half_step(2 * p, 0)
            half_step(2 * p + 1, 1)
            return carry

        lax.fori_loop(0, n_chunks // 2, do_pair, 0)

        for b in range(2):
            pltpu.make_async_copy(
                rows[b], out_hbm.at[pl.ds(base, _CHUNK)], ssem[b]
            ).wait()

    return sc_kernel


def kernel(x, table):
    b, s = x.shape
    batch = b * s
    flat_idx = x.reshape(batch).astype(jnp.int32)
    out = _make_sc_gather(batch)(flat_idx, table)
    return out.reshape(b, s, D_MODEL)
